# baseline (speedup 1.0000x reference)
import jax
import jax.numpy as jnp
from jax import lax
from jax.experimental import pallas as pl
from jax.experimental.pallas import tpu as pltpu
from jax.experimental.pallas import tpu_sc as plsc

MAX_DEEP_C = 30
HIDDEN_C = 1024
TOKENS_C = 4096 * 20
NC, NS, L = 2, 16, 16
NW = NC * NS
TPW = TOKENS_C // NW
CHUNK = 16
PAIRS = TPW // (2 * CHUNK)


def _body(loc_hbm, table_hbm, out_hbm, table_v, loc_v, buf0, buf1, sem0, sem1):
    wid = lax.axis_index("s") * NC + lax.axis_index("c")
    base = wid * TPW

    pltpu.sync_copy(table_hbm, table_v)
    pltpu.sync_copy(loc_hbm.at[pl.ds(base * 1, TPW)], loc_v)

    lanes = lax.iota(jnp.int32, L)
    dnums = lax.GatherDimensionNumbers(
        offset_dims=(), collapsed_slice_dims=(0,), start_index_map=(0,)
    )

    def splat(vec, t):
        idx = jnp.zeros((L, 1), jnp.int32) + t
        return lax.gather(
            vec, idx, dnums, slice_sizes=(1,),
            mode=lax.GatherScatterMode.PROMISE_IN_BOUNDS,
        )

    def do_chunk(c, buf):
        locs = loc_v[pl.ds(pl.multiple_of(c * CHUNK, CHUNK), L)]
        p = locs / 10.0
        xi = p.astype(jnp.int32)
        xi = jnp.minimum(jnp.maximum(xi, 0), MAX_DEEP_C - 1)
        pf = xi.astype(jnp.float32)
        w = (pf - pf) + (pf - pf)
        offv = xi * HIDDEN_C

        def tok(t, carry):
            row = splat(offv, t) + lanes
            w_t = splat(w, t)
            for j in range(HIDDEN_C // L):
                vals = plsc.load_gather(table_v, [row + j * L])
                buf[pl.ds(t * HIDDEN_C + j * L, L)] = w_t * vals
            return carry

        lax.fori_loop(0, L, tok, 0)

    def out_slice(c):
        return out_hbm.at[pl.ds((base + c * CHUNK) * HIDDEN_C, CHUNK * HIDDEN_C)]

    def step(k, carry):
        c0 = 2 * k
        c1 = 2 * k + 1

        @pl.when(k > 0)
        def _():
            pltpu.make_async_copy(buf0, out_slice(c0), sem0).wait()

        do_chunk(c0, buf0)
        pltpu.async_copy(buf0, out_slice(c0), sem0)

        @pl.when(k > 0)
        def _():
            pltpu.make_async_copy(buf1, out_slice(c1), sem1).wait()

        do_chunk(c1, buf1)
        pltpu.async_copy(buf1, out_slice(c1), sem1)
        return carry

    lax.fori_loop(0, PAIRS, step, 0)
    pltpu.make_async_copy(buf0, out_slice(0), sem0).wait()
    pltpu.make_async_copy(buf1, out_slice(1), sem1).wait()


def kernel(location, positional):
    max_deep, _, hidden = positional.shape
    bs, d = location.shape
    loc_flat = location.reshape(bs * d)
    table_flat = positional.reshape(max_deep * hidden)

    mesh = plsc.VectorSubcoreMesh(
        core_axis_name="c", subcore_axis_name="s", num_cores=NC, num_subcores=NS
    )
    out_flat = pl.kernel(
        _body,
        out_type=jax.ShapeDtypeStruct((TOKENS_C * HIDDEN_C,), jnp.float32),
        mesh=mesh,
        compiler_params=pltpu.CompilerParams(needs_layout_passes=False),
        scratch_types=[
            pltpu.VMEM((MAX_DEEP_C * HIDDEN_C,), jnp.float32),
            pltpu.VMEM((TPW,), jnp.float32),
            pltpu.VMEM((CHUNK * HIDDEN_C,), jnp.float32),
            pltpu.VMEM((CHUNK * HIDDEN_C,), jnp.float32),
            pltpu.SemaphoreType.DMA,
            pltpu.SemaphoreType.DMA,
        ],
    )(loc_flat, table_flat)
    return out_flat.reshape(bs * d, 1, hidden)


# speedup vs baseline: 5.4258x; 5.4258x over previous
import jax
import jax.numpy as jnp
from jax import lax
from jax.experimental import pallas as pl
from jax.experimental.pallas import tpu as pltpu
from jax.experimental.pallas import tpu_sc as plsc

MAX_DEEP_C = 30
HIDDEN_C = 1024
TOKENS_C = 4096 * 20
NC, NS, L = 2, 16, 16
NW = NC * NS
TPW = TOKENS_C // NW
CHUNK = 16
PAIRS = TPW // (2 * CHUNK)


def _body(loc_hbm, table_hbm, out_hbm, table_v, loc_v, buf0, buf1, sem0, sem1):
    wid = lax.axis_index("s") * NC + lax.axis_index("c")
    base = wid * TPW

    pltpu.sync_copy(table_hbm, table_v)
    pltpu.sync_copy(loc_hbm.at[pl.ds(base * 1, TPW)], loc_v)

    lanes = lax.iota(jnp.int32, L)
    dnums = lax.GatherDimensionNumbers(
        offset_dims=(), collapsed_slice_dims=(0,), start_index_map=(0,)
    )

    def splat(vec, t):
        idx = jnp.zeros((L, 1), jnp.int32) + t
        return lax.gather(
            vec, idx, dnums, slice_sizes=(1,),
            mode=lax.GatherScatterMode.PROMISE_IN_BOUNDS,
        )

    def do_chunk(c, buf):
        locs = loc_v[pl.ds(pl.multiple_of(c * CHUNK, CHUNK), L)]
        p = locs / 10.0
        xi = p.astype(jnp.int32)
        xi = jnp.minimum(jnp.maximum(xi, 0), MAX_DEEP_C - 1)
        pf = xi.astype(jnp.float32)
        w = (pf - pf) + (pf - pf)
        offv = xi * HIDDEN_C

        def tok(t, carry):
            row = splat(offv, t) + lanes
            w_t = splat(w, t)
            tb = t * HIDDEN_C

            @plsc.parallel_loop(0, HIDDEN_C, step=L, unroll=8)
            def _(o):
                vals = plsc.load_gather(table_v, [row + o])
                buf[pl.ds(tb + o, L)] = w_t * vals

            return carry

        lax.fori_loop(0, L, tok, 0)

    def out_slice(c):
        return out_hbm.at[pl.ds((base + c * CHUNK) * HIDDEN_C, CHUNK * HIDDEN_C)]

    def step(k, carry):
        c0 = 2 * k
        c1 = 2 * k + 1

        @pl.when(k > 0)
        def _():
            pltpu.make_async_copy(buf0, out_slice(c0), sem0).wait()

        do_chunk(c0, buf0)
        pltpu.async_copy(buf0, out_slice(c0), sem0)

        @pl.when(k > 0)
        def _():
            pltpu.make_async_copy(buf1, out_slice(c1), sem1).wait()

        do_chunk(c1, buf1)
        pltpu.async_copy(buf1, out_slice(c1), sem1)
        return carry

    lax.fori_loop(0, PAIRS, step, 0)
    pltpu.make_async_copy(buf0, out_slice(0), sem0).wait()
    pltpu.make_async_copy(buf1, out_slice(1), sem1).wait()


def kernel(location, positional):
    max_deep, _, hidden = positional.shape
    bs, d = location.shape
    loc_flat = location.reshape(bs * d)
    table_flat = positional.reshape(max_deep * hidden)

    mesh = plsc.VectorSubcoreMesh(
        core_axis_name="c", subcore_axis_name="s", num_cores=NC, num_subcores=NS
    )
    out_flat = pl.kernel(
        _body,
        out_type=jax.ShapeDtypeStruct((TOKENS_C * HIDDEN_C,), jnp.float32),
        mesh=mesh,
        compiler_params=pltpu.CompilerParams(needs_layout_passes=False),
        scratch_types=[
            pltpu.VMEM((MAX_DEEP_C * HIDDEN_C,), jnp.float32),
            pltpu.VMEM((TPW,), jnp.float32),
            pltpu.VMEM((CHUNK * HIDDEN_C,), jnp.float32),
            pltpu.VMEM((CHUNK * HIDDEN_C,), jnp.float32),
            pltpu.SemaphoreType.DMA,
            pltpu.SemaphoreType.DMA,
        ],
    )(loc_flat, table_flat)
    return out_flat.reshape(bs * d, 1, hidden)
